# baseline (speedup 1.0000x reference)
import functools

import numpy as np
import jax
import jax.numpy as jnp
from jax import lax
from jax.experimental import pallas as pl
from jax.experimental.pallas import tpu as pltpu
from jax.experimental.pallas import tpu_sc as plsc

_BINS = 10
_NC, _NS = 2, 16
_NW = _NC * _NS
_B, _C = 100000, 80
_N = _B * _C
_PER_W = _N // _NW
_CH = 10000
_NCHUNK = _PER_W // _CH
_VIT = _CH // 16
_TBL = 2 * 11 * 16

_E10 = float(np.float32(1.0) + np.float32(1e-06))


def _sc_body(pred_hbm, targ_hbm, lw_hbm, out_hbm,
             pv0, tv0, lv0, pv1, tv1, lv1, edg, acc, s0, s1):
    wid = lax.axis_index("c") * _NS + lax.axis_index("s")
    base = wid * _PER_W
    zeros16 = jnp.zeros((16,), jnp.float32)
    for j in range(_TBL // 16):
        acc[pl.ds(j * 16, 16)] = zeros16
    lane = lax.iota(jnp.int32, 16)
    ef = lane.astype(jnp.float32) / 10.0
    ef = jnp.where(lane == 10, _E10, jnp.where(lane > 10, 2.0, ef))
    edg[...] = ef
    ones16 = jnp.full((16,), 1.0, jnp.float32)

    def start(c, pv, tv, lv, sem):
        off = base + c * _CH
        pltpu.async_copy(pred_hbm.at[pl.ds(off, _CH)], pv, sem)
        pltpu.async_copy(targ_hbm.at[pl.ds(off, _CH)], tv, sem)
        pltpu.async_copy(lw_hbm.at[pl.ds(off, _CH)], lv, sem)

    def wait(pv, tv, lv, sem):
        pltpu.make_async_copy(pred_hbm.at[pl.ds(0, _CH)], pv, sem).wait()
        pltpu.make_async_copy(targ_hbm.at[pl.ds(0, _CH)], tv, sem).wait()
        pltpu.make_async_copy(lw_hbm.at[pl.ds(0, _CH)], lv, sem).wait()

    def compute(pv, tv, lv):
        @pl.loop(0, _VIT, unroll=5)
        def vec_body(i):
            o = i * 16
            p = pv[pl.ds(o, 16)]
            t = tv[pl.ds(o, 16)].astype(jnp.float32)
            w = lv[pl.ds(o, 16)]
            z = jnp.exp(-jnp.abs(p))
            sig = jnp.where(p >= 0.0, 1.0, z) / (1.0 + z)
            g = jnp.abs(sig - t)
            u = z / (z + 2.0)
            u2 = u * u
            log1pz = (2.0 * u) * (1.0 + u2 * (
                0.3333333333 + u2 * (0.2 + u2 * (0.1428571429 + u2 * 0.1111111111))))
            loss_el = jnp.maximum(p, 0.0) - p * t + log1pz
            valid = w > 0.0
            k0 = jnp.minimum((g * 10.0).astype(jnp.int32), 9)
            e_lo = plsc.load_gather(edg, [k0])
            e_hi = plsc.load_gather(edg, [k0 + 1])
            idx = k0 + jnp.where(g >= e_hi, 1, 0) - jnp.where(g < e_lo, 1, 0)
            idx = jnp.where(valid, idx, _BINS)
            sidx = idx * 16 + lane
            plsc.addupdate_scatter(acc, [sidx], jnp.where(valid, loss_el, 0.0))
            plsc.addupdate_scatter(acc, [sidx + 176], ones16)

    start(0, pv0, tv0, lv0, s0)

    @pl.loop(0, _NCHUNK - 1, step=2)
    def outer(c0):
        wait(pv0, tv0, lv0, s0)
        start(c0 + 1, pv1, tv1, lv1, s1)
        compute(pv0, tv0, lv0)
        wait(pv1, tv1, lv1, s1)
        start(c0 + 2, pv0, tv0, lv0, s0)
        compute(pv1, tv1, lv1)

    wait(pv0, tv0, lv0, s0)
    compute(pv0, tv0, lv0)

    pltpu.sync_copy(acc, out_hbm.at[wid])


_sc_pass = functools.partial(
    pl.kernel,
    mesh=plsc.VectorSubcoreMesh(core_axis_name="c", subcore_axis_name="s"),
    out_type=jax.ShapeDtypeStruct((_NW, _TBL), jnp.float32),
    compiler_params=pltpu.CompilerParams(needs_layout_passes=False),
    scratch_types=[
        pltpu.VMEM((_CH,), jnp.float32),
        pltpu.VMEM((_CH,), jnp.int32),
        pltpu.VMEM((_CH,), jnp.float32),
        pltpu.VMEM((_CH,), jnp.float32),
        pltpu.VMEM((_CH,), jnp.int32),
        pltpu.VMEM((_CH,), jnp.float32),
        pltpu.VMEM((16,), jnp.float32),
        pltpu.VMEM((_TBL,), jnp.float32),
        pltpu.SemaphoreType.DMA,
        pltpu.SemaphoreType.DMA,
    ],
)(_sc_body)


def _finish_body(part_ref, out_ref):
    part = part_ref[...]
    col = jnp.sum(part, axis=0, keepdims=True)
    s_tot = jnp.float32(0.0)
    n = jnp.float32(0.0)
    for b in range(_BINS):
        sb = jnp.sum(col[:, 16 * b:16 * (b + 1)])
        cb = jnp.sum(col[:, 176 + 16 * b:176 + 16 * (b + 1)])
        ne = cb > 0.0
        s_tot = s_tot + jnp.where(ne, sb / jnp.maximum(cb, 1.0), 0.0)
        n = n + jnp.where(ne, 1.0, 0.0)
    out_ref[0, 0] = jnp.where(n > 0.0, s_tot / n, 0.0)


_finish = pl.pallas_call(
    _finish_body,
    out_shape=jax.ShapeDtypeStruct((1, 1), jnp.float32),
    out_specs=pl.BlockSpec(memory_space=pltpu.SMEM),
)


def kernel(pred, target, label_weight):
    p = pred.reshape(-1)
    t = target.reshape(-1)
    w = label_weight.reshape(-1)
    part = _sc_pass(p, t, w)
    return _finish(part)[0, 0]


# speedup vs baseline: 6.7330x; 6.7330x over previous
import functools

import numpy as np
import jax
import jax.numpy as jnp
from jax import lax
from jax.experimental import pallas as pl
from jax.experimental.pallas import tpu as pltpu
from jax.experimental.pallas import tpu_sc as plsc

_BINS = 10
_NC, _NS = 2, 16
_NW = _NC * _NS
_B, _C = 100000, 80
_RT = _B // 8
_TCBLK = 250
_TBL = 2 * 11 * 16

_CT = 10
_CHW = _CT * 1024
_NCH = _RT // _CT

_E10 = float(np.float32(1.0) + np.float32(1e-06))


_BR = 2048
_NGRID = (_B + _BR - 1) // _BR


def _dense_body(pred_ref, targ_ref, lw_ref, pk_ref):
    i = pl.program_id(0)
    p = pred_ref[...].T
    t = targ_ref[...].T.astype(jnp.float32)
    w = lw_ref[...].T
    z = jnp.exp(-jnp.abs(p))
    sig = jnp.where(p >= 0.0, 1.0, z) / (1.0 + z)
    g = jnp.abs(sig - t)
    loss_el = jnp.maximum(p, 0.0) - p * t + jnp.log1p(z)
    rows = jax.lax.broadcasted_iota(jnp.int32, p.shape, 0) + i * _BR
    valid = (w > 0.0) & (rows < _B)
    k0 = jnp.minimum((g * 10.0).astype(jnp.int32), 9)
    e_lo = k0.astype(jnp.float32) / 10.0
    e_hi = jnp.where(k0 == 9, _E10, (k0 + 1).astype(jnp.float32) / 10.0)
    idx = k0 + jnp.where(g >= e_hi, 1, 0) - jnp.where(g < e_lo, 1, 0)
    idx = jnp.where(valid, idx, _BINS)
    vbits = jax.lax.bitcast_convert_type(jnp.where(valid, loss_el, 0.0), jnp.int32)
    packed = (vbits & ~15) | idx
    pk_ref[:, :, 0:80] = packed.reshape(_BR // 8, 8, 80)


_dense = pl.pallas_call(
    _dense_body,
    grid=(_NGRID,),
    in_specs=[
        pl.BlockSpec((_C, _BR), lambda i: (0, i)),
        pl.BlockSpec((_C, _BR), lambda i: (0, i)),
        pl.BlockSpec((_C, _BR), lambda i: (0, i)),
    ],
    out_specs=pl.BlockSpec((_BR // 8, 8, 128), lambda i: (i, 0, 0)),
    out_shape=jax.ShapeDtypeStruct((_RT, 8, 128), jnp.int32),
)


def _sc_body(pk_hbm, out_hbm, iv0, iv1, acc, st, s0, s1):
    wid = lax.axis_index("c") * _NS + lax.axis_index("s")
    zeros16 = jnp.zeros((16,), jnp.float32)
    for j in range(_TBL // 16):
        acc[pl.ds(j * 16, 16)] = zeros16
    ones16 = jnp.full((16,), 1.0, jnp.float32)
    lane = lax.iota(jnp.int32, 16)

    nch_w = (_NCH - wid + _NW - 1) // _NW

    def start(j, iv, sem):
        off = (wid + j * _NW) * _CHW
        pltpu.async_copy(pk_hbm.at[pl.ds(off, _CHW)], iv, sem)

    def wait(iv, sem):
        pltpu.make_async_copy(pk_hbm.at[pl.ds(0, _CHW)], iv, sem).wait()

    def compute(iv):
        @plsc.parallel_loop(0, _CT * 8, unroll=4)
        def vec_body(row):
            base_o = row * 128
            for k in range(5):
                o = base_o + k * 16
                w = iv[pl.ds(o, 16)]
                si = (w & 15) * 16 + lane
                v = plsc.bitcast(w & ~15, jnp.float32)
                plsc.addupdate_scatter(acc, [si], v)
                plsc.addupdate_scatter(acc, [si + 176], ones16)

    start(0, iv0, s0)

    def pair_body(jj, _):
        j0 = jj * 2
        wait(iv0, s0)

        @pl.when(j0 + 1 < nch_w)
        def _():
            start(j0 + 1, iv1, s1)

        compute(iv0)

        @pl.when(j0 + 1 < nch_w)
        def _():
            wait(iv1, s1)

            @pl.when(j0 + 2 < nch_w)
            def _():
                start(j0 + 2, iv0, s0)

            compute(iv1)

        return 0

    lax.fori_loop(0, (nch_w + 1) // 2, pair_body, 0)

    for j in range(_TBL // 16):
        st[j // 8, pl.ds(16 * (j % 8), 16)] = acc[pl.ds(16 * j, 16)]
    pltpu.sync_copy(st, out_hbm.at[wid])


_sc_hist = functools.partial(
    pl.kernel,
    mesh=plsc.VectorSubcoreMesh(core_axis_name="c", subcore_axis_name="s"),
    out_type=jax.ShapeDtypeStruct((_NW, 8, 128), jnp.float32),
    compiler_params=pltpu.CompilerParams(needs_layout_passes=False),
    scratch_types=[
        pltpu.VMEM((_CHW,), jnp.int32),
        pltpu.VMEM((_CHW,), jnp.int32),
        pltpu.VMEM((_TBL,), jnp.float32),
        pltpu.VMEM((8, 128), jnp.float32),
        pltpu.SemaphoreType.DMA,
        pltpu.SemaphoreType.DMA,
    ],
)(_sc_body)


def _finish_body(part_ref, out_ref):
    part = part_ref[...]
    col = jnp.sum(part, axis=0)
    s_tot = jnp.float32(0.0)
    n = jnp.float32(0.0)
    for b in range(_BINS):
        ws = 16 * b
        wc = 176 + 16 * b
        sb = jnp.sum(col[ws // 128, ws % 128:ws % 128 + 16])
        cb = jnp.sum(col[wc // 128, wc % 128:wc % 128 + 16])
        ne = cb > 0.0
        s_tot = s_tot + jnp.where(ne, sb / jnp.maximum(cb, 1.0), 0.0)
        n = n + jnp.where(ne, 1.0, 0.0)
    out_ref[0, 0] = jnp.where(n > 0.0, s_tot / n, 0.0)


_finish = pl.pallas_call(
    _finish_body,
    out_shape=jax.ShapeDtypeStruct((1, 1), jnp.float32),
    out_specs=pl.BlockSpec(memory_space=pltpu.SMEM),
)


def kernel(pred, target, label_weight):
    packed = _dense(pred.T, target.T, label_weight.T)
    part = _sc_hist(packed.reshape(-1))
    return _finish(part)[0, 0]


# speedup vs baseline: 8.9893x; 1.3351x over previous
import functools

import numpy as np
import jax
import jax.numpy as jnp
from jax import lax
from jax.experimental import pallas as pl
from jax.experimental.pallas import tpu as pltpu
from jax.experimental.pallas import tpu_sc as plsc

_BINS = 10
_NC, _NS = 2, 16
_NW = _NC * _NS
_B, _C = 100000, 80
_RT = _B // 8
_TCBLK = 250
_TBL = 2 * 11 * 16

_CT = 25
_CHW = _CT * 1024
_NCH = _RT // _CT

_E10 = float(np.float32(1.0) + np.float32(1e-06))


_BR = 2048
_G1, _G2 = 25, 24
_H1, _H2 = _G1 * _BR, _B - _G1 * _BR
_RT1, _RT2 = _H1 // 8, _H2 // 8


def _make_dense_body(off, nrows):
  def _dense_body(pred_ref, targ_ref, lw_ref, pk_ref):
    i = pl.program_id(0) + off
    p = pred_ref[...]
    t = targ_ref[...].astype(jnp.float32)
    w = lw_ref[...]
    z = jnp.exp(-jnp.abs(p))
    sig = jnp.where(p >= 0.0, 1.0, z) / (1.0 + z)
    g = jnp.abs(sig - t)
    u = z / (z + 2.0)
    u2 = u * u
    log1pz = (2.0 * u) * (1.0 + u2 * (
        0.3333333333 + u2 * (0.2 + u2 * (0.1428571429 + u2 * 0.1111111111))))
    loss_el = jnp.maximum(p, 0.0) - p * t + log1pz
    rows = jax.lax.broadcasted_iota(jnp.int32, p.shape, 1) + i * _BR
    valid = (w > 0.0) & (rows < nrows)
    k0 = jnp.minimum((g * 10.0).astype(jnp.int32), 9)
    e_lo = k0.astype(jnp.float32) / 10.0
    e_hi = jnp.where(k0 == 9, _E10, (k0 + 1).astype(jnp.float32) / 10.0)
    idx = k0 + jnp.where(g >= e_hi, 1, 0) - jnp.where(g < e_lo, 1, 0)
    idx = jnp.where(valid, idx, _BINS)
    vbits = jax.lax.bitcast_convert_type(jnp.where(valid, loss_el, 0.0), jnp.int32)
    packed = (vbits & ~15) | idx
    pk_ref[:, :, 0:80] = packed.T.reshape(_BR // 8, 8, 80)

  return _dense_body


def _make_dense(off, ngrid, nrows, ntiles):
    def imap(i, off=off):
        return (0, i + off)

    return pl.pallas_call(
        _make_dense_body(off, nrows),
        grid=(ngrid,),
        in_specs=[
            pl.BlockSpec((_C, _BR), imap),
            pl.BlockSpec((_C, _BR), imap),
            pl.BlockSpec((_C, _BR), imap),
        ],
        out_specs=pl.BlockSpec((_BR // 8, 8, 128), lambda i: (i, 0, 0)),
        out_shape=jax.ShapeDtypeStruct((ntiles, 8, 128), jnp.int32),
    )


_dense1 = _make_dense(0, _G1, _B, _RT1)
_dense2 = _make_dense(_G1, _G2, _B, _RT2)


def _make_sc_body(nch):
  def _sc_body(pk_hbm, out_hbm, iv0, iv1, acc, st, s0, s1):
    wid = lax.axis_index("c") * _NS + lax.axis_index("s")
    zeros16 = jnp.zeros((16,), jnp.float32)
    for j in range(_TBL // 16):
        acc[pl.ds(j * 16, 16)] = zeros16
    ones16 = jnp.full((16,), 1.0, jnp.float32)
    lane = lax.iota(jnp.int32, 16)

    nch_w = (nch - wid + _NW - 1) // _NW

    def start(j, iv, sem):
        off = (wid + j * _NW) * _CHW
        pltpu.async_copy(pk_hbm.at[pl.ds(off, _CHW)], iv, sem)

    def wait(iv, sem):
        pltpu.make_async_copy(pk_hbm.at[pl.ds(0, _CHW)], iv, sem).wait()

    def compute(iv):
        @plsc.parallel_loop(0, _CT * 8, unroll=4)
        def vec_body(row):
            base_o = row * 128
            for k in range(5):
                o = base_o + k * 16
                w = iv[pl.ds(o, 16)]
                si = (w & 15) * 16 + lane
                v = plsc.bitcast(w & ~15, jnp.float32)
                plsc.addupdate_scatter(acc, [si], v)
                plsc.addupdate_scatter(acc, [si + 176], ones16)

    start(0, iv0, s0)

    def pair_body(jj, _):
        j0 = jj * 2
        wait(iv0, s0)

        @pl.when(j0 + 1 < nch_w)
        def _():
            start(j0 + 1, iv1, s1)

        compute(iv0)

        @pl.when(j0 + 1 < nch_w)
        def _():
            wait(iv1, s1)

            @pl.when(j0 + 2 < nch_w)
            def _():
                start(j0 + 2, iv0, s0)

            compute(iv1)

        return 0

    lax.fori_loop(0, (nch_w + 1) // 2, pair_body, 0)

    for j in range(_TBL // 16):
        st[j // 8, pl.ds(16 * (j % 8), 16)] = acc[pl.ds(16 * j, 16)]
    pltpu.sync_copy(st, out_hbm.at[wid])

  return _sc_body


def _make_sc_hist(nch):
    return functools.partial(
        pl.kernel,
        mesh=plsc.VectorSubcoreMesh(core_axis_name="c", subcore_axis_name="s"),
        out_type=jax.ShapeDtypeStruct((_NW, 8, 128), jnp.float32),
        compiler_params=pltpu.CompilerParams(needs_layout_passes=False),
        scratch_types=[
            pltpu.VMEM((_CHW,), jnp.int32),
            pltpu.VMEM((_CHW,), jnp.int32),
            pltpu.VMEM((_TBL,), jnp.float32),
            pltpu.VMEM((8, 128), jnp.float32),
            pltpu.SemaphoreType.DMA,
            pltpu.SemaphoreType.DMA,
        ],
    )(_make_sc_body(nch))


_sc_hist1 = _make_sc_hist(_RT1 // _CT)
_sc_hist2 = _make_sc_hist(_RT2 // _CT)


def _finish_body(p1_ref, p2_ref, out_ref):
    col = jnp.sum(p1_ref[...], axis=0) + jnp.sum(p2_ref[...], axis=0)
    s_tot = jnp.float32(0.0)
    n = jnp.float32(0.0)
    for b in range(_BINS):
        ws = 16 * b
        wc = 176 + 16 * b
        sb = jnp.sum(col[ws // 128, ws % 128:ws % 128 + 16])
        cb = jnp.sum(col[wc // 128, wc % 128:wc % 128 + 16])
        ne = cb > 0.0
        s_tot = s_tot + jnp.where(ne, sb / jnp.maximum(cb, 1.0), 0.0)
        n = n + jnp.where(ne, 1.0, 0.0)
    out_ref[0, 0] = jnp.where(n > 0.0, s_tot / n, 0.0)


_finish = pl.pallas_call(
    _finish_body,
    out_shape=jax.ShapeDtypeStruct((1, 1), jnp.float32),
    out_specs=pl.BlockSpec(memory_space=pltpu.SMEM),
)


def kernel(pred, target, label_weight):
    pT = pred.T
    tT = target.T
    wT = label_weight.T
    pk1 = _dense1(pT, tT, wT)
    part1 = _sc_hist1(pk1.reshape(-1))
    pk2 = _dense2(pT, tT, wT)
    part2 = _sc_hist2(pk2.reshape(-1))
    return _finish(part1, part2)[0, 0]
